# baseline (speedup 1.0000x reference)
import functools

import jax
import jax.numpy as jnp
from jax import lax
from jax.experimental import pallas as pl
from jax.experimental.pallas import tpu as pltpu
from jax.experimental.pallas import tpu_sc as plsc

NUM_NODES = 1000000
EMBED_DIM = 64
BATCH = 16384
PAIR = 2 * EMBED_DIM

NC = 2
NS = 16
LANES = 16
NW = NC * NS
BPW = BATCH // NW
CHUNK = 128
NCHUNK = BPW // CHUNK
HALVES = 2
ROWS_H = BPW // HALVES
CH_H = NCHUNK // HALVES
GROUPS_H = ROWS_H // LANES

_mesh = plsc.VectorSubcoreMesh(
    core_axis_name="c", subcore_axis_name="s", num_cores=NC, num_subcores=NS
)


@functools.partial(
    pl.kernel,
    out_type=jax.ShapeDtypeStruct((BATCH,), jnp.float32),
    mesh=_mesh,
    scratch_types=[
        pltpu.VMEM((BPW, 3), jnp.int32),
        pltpu.VMEM((NCHUNK, CHUNK), jnp.int32),
        pltpu.VMEM((NCHUNK, CHUNK), jnp.int32),
        pltpu.VMEM((BPW,), jnp.int32),
        pltpu.VMEM((BPW,), jnp.int32),
        pltpu.VMEM((ROWS_H, PAIR), jnp.float32),
        pltpu.VMEM((ROWS_H, PAIR), jnp.float32),
        pltpu.VMEM((BPW,), jnp.float32),
        pltpu.SemaphoreType.DMA,
    ],
    compiler_params=pltpu.CompilerParams(
        needs_layout_passes=False, use_tc_tiling_on_sc=False),
)
def _score_kernel(tri_hbm, emb2_hbm, out_hbm,
                  tri_v, sidx_v, oidx_v, spar_v, opar_v,
                  srows_v, orows_v, out_v, sem):
    wid = lax.axis_index("s") * NC + lax.axis_index("c")
    base = wid * BPW

    pltpu.sync_copy(tri_hbm.at[pl.ds(base, BPW)], tri_v)

    lane = jnp.arange(LANES, dtype=jnp.int32)
    col0 = jnp.zeros((LANES,), jnp.int32)
    col1 = jnp.ones((LANES,), jnp.int32)

    for g in range(BPW // LANES):
        rows = g * LANES + lane
        s = plsc.load_gather(tri_v, [rows, col0])
        o = plsc.load_gather(tri_v, [rows, col1])
        j, off = divmod(g, CHUNK // LANES)
        sidx_v[j, pl.ds(off * LANES, LANES)] = s >> 1
        oidx_v[j, pl.ds(off * LANES, LANES)] = o >> 1
        spar_v[pl.ds(g * LANES, LANES)] = (s & 1) * EMBED_DIM
        opar_v[pl.ds(g * LANES, LANES)] = (o & 1) * EMBED_DIM

    for h in range(HALVES):
        copies = []
        for j in range(CH_H):
            copies.append(pltpu.async_copy(
                emb2_hbm.at[sidx_v.at[h * CH_H + j]],
                srows_v.at[pl.ds(j * CHUNK, CHUNK)], sem))
            copies.append(pltpu.async_copy(
                emb2_hbm.at[oidx_v.at[h * CH_H + j]],
                orows_v.at[pl.ds(j * CHUNK, CHUNK)], sem))
        for c in copies:
            c.wait()

        def group_body(g, carry):
            loc = g * LANES + lane
            sp = spar_v[pl.ds(h * ROWS_H + g * LANES, LANES)]
            op = opar_v[pl.ds(h * ROWS_H + g * LANES, LANES)]
            acc = jnp.zeros((LANES,), jnp.float32)
            for d in range(EMBED_DIM):
                sv = plsc.load_gather(srows_v, [loc, sp + d])
                ov = plsc.load_gather(orows_v, [loc, op + d])
                acc = acc + sv * ov
            out_v[pl.ds(h * ROWS_H + g * LANES, LANES)] = acc
            return carry

        lax.fori_loop(0, GROUPS_H, group_body, 0)

    pltpu.sync_copy(out_v, out_hbm.at[pl.ds(base, BPW)])


def kernel(triples, entity_embedding):
    emb2 = jnp.concatenate(
        [entity_embedding[0::2, :], entity_embedding[1::2, :]], axis=1)
    return _score_kernel(triples, emb2)


# speedup vs baseline: 3.6183x; 3.6183x over previous
import functools

import jax
import jax.numpy as jnp
from jax import lax
from jax.experimental import pallas as pl
from jax.experimental.pallas import tpu as pltpu
from jax.experimental.pallas import tpu_sc as plsc

NUM_NODES = 1000000
EMBED_DIM = 64
BATCH = 16384
PAIR = 2 * EMBED_DIM

NC = 2
NS = 16
LANES = 16
NW = NC * NS
BPW = BATCH // NW
CHUNK = 128
NCHUNK = BPW // CHUNK
HALVES = 2
ROWS_H = BPW // HALVES
CH_H = NCHUNK // HALVES
GROUPS_H = ROWS_H // LANES

ASTEPS = 3907
OUT_ROWS = ASTEPS * 128


def _relayout_body(x1_ref, x2_ref, out_ref):
    x1 = x1_ref[...]
    x2 = x2_ref[...]
    eye = jnp.eye(EMBED_DIM, dtype=jnp.float32)
    t1 = jax.lax.dot_general(x1, eye, (((0,), (0,)), ((), ())),
                             preferred_element_type=jnp.float32)
    t2 = jax.lax.dot_general(x2, eye, (((0,), (0,)), ((), ())),
                             preferred_element_type=jnp.float32)
    out_ref[...] = jnp.concatenate([t1, t2], axis=1)


_relayout = pl.pallas_call(
    _relayout_body,
    grid=(ASTEPS,),
    in_specs=[
        pl.BlockSpec((EMBED_DIM, 128), lambda i: (0, 2 * i)),
        pl.BlockSpec((EMBED_DIM, 128), lambda i: (0, 2 * i + 1)),
    ],
    out_specs=pl.BlockSpec((128, PAIR), lambda i: (i, 0)),
    out_shape=jax.ShapeDtypeStruct((OUT_ROWS, PAIR), jnp.float32),
)

_mesh = plsc.VectorSubcoreMesh(
    core_axis_name="c", subcore_axis_name="s", num_cores=NC, num_subcores=NS
)


@functools.partial(
    pl.kernel,
    out_type=jax.ShapeDtypeStruct((BATCH,), jnp.float32),
    mesh=_mesh,
    scratch_types=[
        pltpu.VMEM((BPW, 3), jnp.int32),
        pltpu.VMEM((NCHUNK, CHUNK), jnp.int32),
        pltpu.VMEM((NCHUNK, CHUNK), jnp.int32),
        pltpu.VMEM((BPW,), jnp.int32),
        pltpu.VMEM((BPW,), jnp.int32),
        pltpu.VMEM((ROWS_H, PAIR), jnp.float32),
        pltpu.VMEM((ROWS_H, PAIR), jnp.float32),
        pltpu.VMEM((BPW,), jnp.float32),
        pltpu.SemaphoreType.DMA,
    ],
    compiler_params=pltpu.CompilerParams(
        needs_layout_passes=False, use_tc_tiling_on_sc=False),
)
def _score_kernel(tri_hbm, emb2_hbm, out_hbm,
                  tri_v, sidx_v, oidx_v, spar_v, opar_v,
                  srows_v, orows_v, out_v, sem):
    wid = lax.axis_index("s") * NC + lax.axis_index("c")
    base = wid * BPW

    pltpu.sync_copy(tri_hbm.at[pl.ds(base, BPW)], tri_v)

    lane = jnp.arange(LANES, dtype=jnp.int32)
    col0 = jnp.zeros((LANES,), jnp.int32)
    col1 = jnp.ones((LANES,), jnp.int32)

    for g in range(BPW // LANES):
        rows = g * LANES + lane
        s = plsc.load_gather(tri_v, [rows, col0])
        o = plsc.load_gather(tri_v, [rows, col1])
        j, off = divmod(g, CHUNK // LANES)
        sidx_v[j, pl.ds(off * LANES, LANES)] = ((s >> 8) << 7) | (s & 127)
        oidx_v[j, pl.ds(off * LANES, LANES)] = ((o >> 8) << 7) | (o & 127)
        spar_v[pl.ds(g * LANES, LANES)] = ((s >> 7) & 1) * EMBED_DIM
        opar_v[pl.ds(g * LANES, LANES)] = ((o >> 7) & 1) * EMBED_DIM

    for h in range(HALVES):
        copies = []
        for j in range(CH_H):
            copies.append(pltpu.async_copy(
                emb2_hbm.at[sidx_v.at[h * CH_H + j]],
                srows_v.at[pl.ds(j * CHUNK, CHUNK)], sem))
            copies.append(pltpu.async_copy(
                emb2_hbm.at[oidx_v.at[h * CH_H + j]],
                orows_v.at[pl.ds(j * CHUNK, CHUNK)], sem))
        for c in copies:
            c.wait()

        def group_body(g, carry):
            loc = g * LANES + lane
            sp = spar_v[pl.ds(h * ROWS_H + g * LANES, LANES)]
            op = opar_v[pl.ds(h * ROWS_H + g * LANES, LANES)]
            acc = jnp.zeros((LANES,), jnp.float32)
            for d in range(EMBED_DIM):
                sv = plsc.load_gather(srows_v, [loc, sp + d])
                ov = plsc.load_gather(orows_v, [loc, op + d])
                acc = acc + sv * ov
            out_v[pl.ds(h * ROWS_H + g * LANES, LANES)] = acc
            return carry

        lax.fori_loop(0, GROUPS_H, group_body, 0)

    pltpu.sync_copy(out_v, out_hbm.at[pl.ds(base, BPW)])


def kernel(triples, entity_embedding):
    emb2 = _relayout(entity_embedding.T, entity_embedding.T)
    return _score_kernel(triples, emb2)


# speedup vs baseline: 21.9007x; 6.0528x over previous
import functools

import jax
import jax.numpy as jnp
from jax import lax
from jax.experimental import pallas as pl
from jax.experimental.pallas import tpu as pltpu
from jax.experimental.pallas import tpu_sc as plsc

NUM_NODES = 1000000
EMBED_DIM = 64
BATCH = 16384
PAIR = 2 * EMBED_DIM

NC = 2
NS = 16
LANES = 16
NW = NC * NS
BPW = BATCH // NW
CHUNK = 128
NCHUNK = BPW // CHUNK
HALVES = 2
ROWS_H = BPW // HALVES
CH_H = NCHUNK // HALVES
GROUPS_H = ROWS_H // LANES

ABLK = 4096
ASTEPS = -(-NUM_NODES // ABLK)
OUT_ROWS = ASTEPS * (ABLK // 2)


def _relayout_body(x_ref, out_ref):
    x = x_ref[...]
    eye = jnp.eye(EMBED_DIM, dtype=jnp.float32)
    t = jax.lax.dot_general(x, eye, (((0,), (0,)), ((), ())),
                            preferred_element_type=jnp.float32)
    packed = [
        jnp.concatenate([t[a * 256:a * 256 + 128],
                         t[a * 256 + 128:a * 256 + 256]], axis=1)
        for a in range(ABLK // 256)
    ]
    out_ref[...] = jnp.concatenate(packed, axis=0)


_relayout = pl.pallas_call(
    _relayout_body,
    grid=(ASTEPS,),
    in_specs=[
        pl.BlockSpec((EMBED_DIM, ABLK), lambda i: (0, i)),
    ],
    out_specs=pl.BlockSpec((ABLK // 2, PAIR), lambda i: (i, 0)),
    out_shape=jax.ShapeDtypeStruct((OUT_ROWS, PAIR), jnp.float32),
)

_mesh = plsc.VectorSubcoreMesh(
    core_axis_name="c", subcore_axis_name="s", num_cores=NC, num_subcores=NS
)


@functools.partial(
    pl.kernel,
    out_type=jax.ShapeDtypeStruct((BATCH,), jnp.float32),
    mesh=_mesh,
    scratch_types=[
        pltpu.VMEM((BPW, 3), jnp.int32),
        pltpu.VMEM((NCHUNK, CHUNK), jnp.int32),
        pltpu.VMEM((NCHUNK, CHUNK), jnp.int32),
        pltpu.VMEM((BPW,), jnp.int32),
        pltpu.VMEM((BPW,), jnp.int32),
        pltpu.VMEM((ROWS_H, PAIR), jnp.float32),
        pltpu.VMEM((ROWS_H, PAIR), jnp.float32),
        pltpu.VMEM((BPW,), jnp.float32),
        pltpu.SemaphoreType.DMA,
    ],
    compiler_params=pltpu.CompilerParams(
        needs_layout_passes=False, use_tc_tiling_on_sc=False),
)
def _score_kernel(tri_hbm, emb2_hbm, out_hbm,
                  tri_v, sidx_v, oidx_v, spar_v, opar_v,
                  srows_v, orows_v, out_v, sem):
    wid = lax.axis_index("s") * NC + lax.axis_index("c")
    base = wid * BPW

    pltpu.sync_copy(tri_hbm.at[pl.ds(base, BPW)], tri_v)

    lane = jnp.arange(LANES, dtype=jnp.int32)
    col0 = jnp.zeros((LANES,), jnp.int32)
    col1 = jnp.ones((LANES,), jnp.int32)

    for g in range(BPW // LANES):
        rows = g * LANES + lane
        s = plsc.load_gather(tri_v, [rows, col0])
        o = plsc.load_gather(tri_v, [rows, col1])
        j, off = divmod(g, CHUNK // LANES)
        sidx_v[j, pl.ds(off * LANES, LANES)] = ((s >> 8) << 7) | (s & 127)
        oidx_v[j, pl.ds(off * LANES, LANES)] = ((o >> 8) << 7) | (o & 127)
        spar_v[pl.ds(g * LANES, LANES)] = ((s >> 7) & 1) * EMBED_DIM
        opar_v[pl.ds(g * LANES, LANES)] = ((o >> 7) & 1) * EMBED_DIM

    for h in range(HALVES):
        copies = []
        for j in range(CH_H):
            copies.append(pltpu.async_copy(
                emb2_hbm.at[sidx_v.at[h * CH_H + j]],
                srows_v.at[pl.ds(j * CHUNK, CHUNK)], sem))
            copies.append(pltpu.async_copy(
                emb2_hbm.at[oidx_v.at[h * CH_H + j]],
                orows_v.at[pl.ds(j * CHUNK, CHUNK)], sem))
        for c in copies:
            c.wait()

        def group_body(g, carry):
            loc = g * LANES + lane
            sp = spar_v[pl.ds(h * ROWS_H + g * LANES, LANES)]
            op = opar_v[pl.ds(h * ROWS_H + g * LANES, LANES)]
            acc = jnp.zeros((LANES,), jnp.float32)
            for d in range(EMBED_DIM):
                sv = plsc.load_gather(srows_v, [loc, sp + d])
                ov = plsc.load_gather(orows_v, [loc, op + d])
                acc = acc + sv * ov
            out_v[pl.ds(h * ROWS_H + g * LANES, LANES)] = acc
            return carry

        lax.fori_loop(0, GROUPS_H, group_body, 0)

    pltpu.sync_copy(out_v, out_hbm.at[pl.ds(base, BPW)])


def kernel(triples, entity_embedding):
    emb2 = _relayout(entity_embedding.T)
    return _score_kernel(triples, emb2)


# speedup vs baseline: 21.9449x; 1.0020x over previous
import functools

import jax
import jax.numpy as jnp
from jax import lax
from jax.experimental import pallas as pl
from jax.experimental.pallas import tpu as pltpu
from jax.experimental.pallas import tpu_sc as plsc

NUM_NODES = 1000000
EMBED_DIM = 64
BATCH = 16384
PAIR = 2 * EMBED_DIM

NC = 2
NS = 16
LANES = 16
NW = NC * NS
BPW = BATCH // NW
CHUNK = 128
NCHUNK = BPW // CHUNK
HALVES = 2
ROWS_H = BPW // HALVES
CH_H = NCHUNK // HALVES
GROUPS_H = ROWS_H // LANES

ABLK = 4096
ASTEPS = -(-NUM_NODES // ABLK)
OUT_ROWS = ASTEPS * (ABLK // 2)


def _relayout_body(x_ref, out_ref):
    x = x_ref[...]
    t = x.T
    packed = [
        jnp.concatenate([t[a * 256:a * 256 + 128],
                         t[a * 256 + 128:a * 256 + 256]], axis=1)
        for a in range(ABLK // 256)
    ]
    out_ref[...] = jnp.concatenate(packed, axis=0)


_relayout = pl.pallas_call(
    _relayout_body,
    grid=(ASTEPS,),
    in_specs=[
        pl.BlockSpec((EMBED_DIM, ABLK), lambda i: (0, i)),
    ],
    out_specs=pl.BlockSpec((ABLK // 2, PAIR), lambda i: (i, 0)),
    out_shape=jax.ShapeDtypeStruct((OUT_ROWS, PAIR), jnp.float32),
)

_mesh = plsc.VectorSubcoreMesh(
    core_axis_name="c", subcore_axis_name="s", num_cores=NC, num_subcores=NS
)


@functools.partial(
    pl.kernel,
    out_type=jax.ShapeDtypeStruct((BATCH,), jnp.float32),
    mesh=_mesh,
    scratch_types=[
        pltpu.VMEM((BPW, 3), jnp.int32),
        pltpu.VMEM((NCHUNK, CHUNK), jnp.int32),
        pltpu.VMEM((NCHUNK, CHUNK), jnp.int32),
        pltpu.VMEM((BPW,), jnp.int32),
        pltpu.VMEM((BPW,), jnp.int32),
        pltpu.VMEM((ROWS_H, PAIR), jnp.float32),
        pltpu.VMEM((ROWS_H, PAIR), jnp.float32),
        pltpu.VMEM((BPW,), jnp.float32),
        pltpu.SemaphoreType.DMA,
    ],
    compiler_params=pltpu.CompilerParams(
        needs_layout_passes=False, use_tc_tiling_on_sc=False),
)
def _score_kernel(tri_hbm, emb2_hbm, out_hbm,
                  tri_v, sidx_v, oidx_v, spar_v, opar_v,
                  srows_v, orows_v, out_v, sem):
    wid = lax.axis_index("s") * NC + lax.axis_index("c")
    base = wid * BPW

    pltpu.sync_copy(tri_hbm.at[pl.ds(base, BPW)], tri_v)

    lane = jnp.arange(LANES, dtype=jnp.int32)
    col0 = jnp.zeros((LANES,), jnp.int32)
    col1 = jnp.ones((LANES,), jnp.int32)

    for g in range(BPW // LANES):
        rows = g * LANES + lane
        s = plsc.load_gather(tri_v, [rows, col0])
        o = plsc.load_gather(tri_v, [rows, col1])
        j, off = divmod(g, CHUNK // LANES)
        sidx_v[j, pl.ds(off * LANES, LANES)] = ((s >> 8) << 7) | (s & 127)
        oidx_v[j, pl.ds(off * LANES, LANES)] = ((o >> 8) << 7) | (o & 127)
        spar_v[pl.ds(g * LANES, LANES)] = ((s >> 7) & 1) * EMBED_DIM
        opar_v[pl.ds(g * LANES, LANES)] = ((o >> 7) & 1) * EMBED_DIM

    for h in range(HALVES):
        copies = []
        for j in range(CH_H):
            copies.append(pltpu.async_copy(
                emb2_hbm.at[sidx_v.at[h * CH_H + j]],
                srows_v.at[pl.ds(j * CHUNK, CHUNK)], sem))
            copies.append(pltpu.async_copy(
                emb2_hbm.at[oidx_v.at[h * CH_H + j]],
                orows_v.at[pl.ds(j * CHUNK, CHUNK)], sem))
        for c in copies:
            c.wait()

        def group_body(g, carry):
            loc = g * LANES + lane
            sp = spar_v[pl.ds(h * ROWS_H + g * LANES, LANES)]
            op = opar_v[pl.ds(h * ROWS_H + g * LANES, LANES)]
            acc = jnp.zeros((LANES,), jnp.float32)
            for d in range(EMBED_DIM):
                sv = plsc.load_gather(srows_v, [loc, sp + d])
                ov = plsc.load_gather(orows_v, [loc, op + d])
                acc = acc + sv * ov
            out_v[pl.ds(h * ROWS_H + g * LANES, LANES)] = acc
            return carry

        lax.fori_loop(0, GROUPS_H, group_body, 0)

    pltpu.sync_copy(out_v, out_hbm.at[pl.ds(base, BPW)])


def kernel(triples, entity_embedding):
    emb2 = _relayout(entity_embedding.T)
    return _score_kernel(triples, emb2)


# speedup vs baseline: 26.4403x; 1.2049x over previous
import functools

import jax
import jax.numpy as jnp
from jax import lax
from jax.experimental import pallas as pl
from jax.experimental.pallas import tpu as pltpu
from jax.experimental.pallas import tpu_sc as plsc

NUM_NODES = 1000000
EMBED_DIM = 64
BATCH = 16384
PAIR = 2 * EMBED_DIM

NC = 2
NS = 16
LANES = 16
NW = NC * NS
BPW = BATCH // NW
CHUNK = 128
NCHUNK = BPW // CHUNK
HALVES = 2
ROWS_H = BPW // HALVES
CH_H = NCHUNK // HALVES
GROUPS_H = ROWS_H // LANES

ABLK = 8192
ASTEPS = -(-NUM_NODES // ABLK)
OUT_ROWS = ASTEPS * (ABLK // 2)


def _relayout_body(x_ref, out_ref):
    x = x_ref[...]
    t = x.T
    packed = [
        jnp.concatenate([t[a * 256:a * 256 + 128],
                         t[a * 256 + 128:a * 256 + 256]], axis=1)
        for a in range(ABLK // 256)
    ]
    out_ref[...] = jnp.concatenate(packed, axis=0)


_relayout = pl.pallas_call(
    _relayout_body,
    grid=(ASTEPS,),
    in_specs=[
        pl.BlockSpec((EMBED_DIM, ABLK), lambda i: (0, i)),
    ],
    out_specs=pl.BlockSpec((ABLK // 2, PAIR), lambda i: (i, 0)),
    out_shape=jax.ShapeDtypeStruct((OUT_ROWS, PAIR), jnp.float32),
)

_mesh = plsc.VectorSubcoreMesh(
    core_axis_name="c", subcore_axis_name="s", num_cores=NC, num_subcores=NS
)


@functools.partial(
    pl.kernel,
    out_type=jax.ShapeDtypeStruct((BATCH,), jnp.float32),
    mesh=_mesh,
    scratch_types=[
        pltpu.VMEM((BPW, 3), jnp.int32),
        pltpu.VMEM((NCHUNK, CHUNK), jnp.int32),
        pltpu.VMEM((NCHUNK, CHUNK), jnp.int32),
        pltpu.VMEM((BPW,), jnp.int32),
        pltpu.VMEM((BPW,), jnp.int32),
        pltpu.VMEM((ROWS_H, PAIR), jnp.float32),
        pltpu.VMEM((ROWS_H, PAIR), jnp.float32),
        pltpu.VMEM((BPW,), jnp.float32),
        pltpu.SemaphoreType.DMA,
    ],
    compiler_params=pltpu.CompilerParams(
        needs_layout_passes=False, use_tc_tiling_on_sc=False),
)
def _score_kernel(tri_hbm, emb2_hbm, out_hbm,
                  tri_v, sidx_v, oidx_v, spar_v, opar_v,
                  srows_v, orows_v, out_v, sem):
    wid = lax.axis_index("s") * NC + lax.axis_index("c")
    base = wid * BPW

    pltpu.sync_copy(tri_hbm.at[pl.ds(base, BPW)], tri_v)

    lane = jnp.arange(LANES, dtype=jnp.int32)
    col0 = jnp.zeros((LANES,), jnp.int32)
    col1 = jnp.ones((LANES,), jnp.int32)

    for g in range(BPW // LANES):
        rows = g * LANES + lane
        s = plsc.load_gather(tri_v, [rows, col0])
        o = plsc.load_gather(tri_v, [rows, col1])
        j, off = divmod(g, CHUNK // LANES)
        sidx_v[j, pl.ds(off * LANES, LANES)] = ((s >> 8) << 7) | (s & 127)
        oidx_v[j, pl.ds(off * LANES, LANES)] = ((o >> 8) << 7) | (o & 127)
        spar_v[pl.ds(g * LANES, LANES)] = ((s >> 7) & 1) * EMBED_DIM
        opar_v[pl.ds(g * LANES, LANES)] = ((o >> 7) & 1) * EMBED_DIM

    for h in range(HALVES):
        copies = []
        for j in range(CH_H):
            copies.append(pltpu.async_copy(
                emb2_hbm.at[sidx_v.at[h * CH_H + j]],
                srows_v.at[pl.ds(j * CHUNK, CHUNK)], sem))
            copies.append(pltpu.async_copy(
                emb2_hbm.at[oidx_v.at[h * CH_H + j]],
                orows_v.at[pl.ds(j * CHUNK, CHUNK)], sem))
        for c in copies:
            c.wait()

        def group_body(g, carry):
            loc = g * LANES + lane
            sp = spar_v[pl.ds(h * ROWS_H + g * LANES, LANES)]
            op = opar_v[pl.ds(h * ROWS_H + g * LANES, LANES)]
            acc = jnp.zeros((LANES,), jnp.float32)
            for d in range(EMBED_DIM):
                sv = plsc.load_gather(srows_v, [loc, sp + d])
                ov = plsc.load_gather(orows_v, [loc, op + d])
                acc = acc + sv * ov
            out_v[pl.ds(h * ROWS_H + g * LANES, LANES)] = acc
            return carry

        lax.fori_loop(0, GROUPS_H, group_body, 0)

    pltpu.sync_copy(out_v, out_hbm.at[pl.ds(base, BPW)])


def kernel(triples, entity_embedding):
    emb2 = _relayout(entity_embedding.T)
    return _score_kernel(triples, emb2)


# speedup vs baseline: 29.2040x; 1.1045x over previous
import functools

import jax
import jax.numpy as jnp
from jax import lax
from jax.experimental import pallas as pl
from jax.experimental.pallas import tpu as pltpu
from jax.experimental.pallas import tpu_sc as plsc

NUM_NODES = 1000000
EMBED_DIM = 64
BATCH = 16384
PAIR = 2 * EMBED_DIM

NC = 2
NS = 16
LANES = 16
NW = NC * NS
BPW = BATCH // NW
CHUNK = 128
NCHUNK = BPW // CHUNK
HALVES = 2
ROWS_H = BPW // HALVES
CH_H = NCHUNK // HALVES
GROUPS_H = ROWS_H // LANES

ABLK = 16384
ASTEPS = -(-NUM_NODES // ABLK)
OUT_ROWS = ASTEPS * (ABLK // 2)


def _relayout_body(x_ref, out_ref):
    x = x_ref[...]
    t = x.T
    packed = [
        jnp.concatenate([t[a * 256:a * 256 + 128],
                         t[a * 256 + 128:a * 256 + 256]], axis=1)
        for a in range(ABLK // 256)
    ]
    out_ref[...] = jnp.concatenate(packed, axis=0)


_relayout = pl.pallas_call(
    _relayout_body,
    grid=(ASTEPS,),
    in_specs=[
        pl.BlockSpec((EMBED_DIM, ABLK), lambda i: (0, i)),
    ],
    out_specs=pl.BlockSpec((ABLK // 2, PAIR), lambda i: (i, 0)),
    out_shape=jax.ShapeDtypeStruct((OUT_ROWS, PAIR), jnp.float32),
)

_mesh = plsc.VectorSubcoreMesh(
    core_axis_name="c", subcore_axis_name="s", num_cores=NC, num_subcores=NS
)


@functools.partial(
    pl.kernel,
    out_type=jax.ShapeDtypeStruct((BATCH,), jnp.float32),
    mesh=_mesh,
    scratch_types=[
        pltpu.VMEM((BPW, 3), jnp.int32),
        pltpu.VMEM((NCHUNK, CHUNK), jnp.int32),
        pltpu.VMEM((NCHUNK, CHUNK), jnp.int32),
        pltpu.VMEM((BPW,), jnp.int32),
        pltpu.VMEM((BPW,), jnp.int32),
        pltpu.VMEM((ROWS_H, PAIR), jnp.float32),
        pltpu.VMEM((ROWS_H, PAIR), jnp.float32),
        pltpu.VMEM((BPW,), jnp.float32),
        pltpu.SemaphoreType.DMA,
    ],
    compiler_params=pltpu.CompilerParams(
        needs_layout_passes=False, use_tc_tiling_on_sc=False),
)
def _score_kernel(tri_hbm, emb2_hbm, out_hbm,
                  tri_v, sidx_v, oidx_v, spar_v, opar_v,
                  srows_v, orows_v, out_v, sem):
    wid = lax.axis_index("s") * NC + lax.axis_index("c")
    base = wid * BPW

    pltpu.sync_copy(tri_hbm.at[pl.ds(base, BPW)], tri_v)

    lane = jnp.arange(LANES, dtype=jnp.int32)
    col0 = jnp.zeros((LANES,), jnp.int32)
    col1 = jnp.ones((LANES,), jnp.int32)

    for g in range(BPW // LANES):
        rows = g * LANES + lane
        s = plsc.load_gather(tri_v, [rows, col0])
        o = plsc.load_gather(tri_v, [rows, col1])
        j, off = divmod(g, CHUNK // LANES)
        sidx_v[j, pl.ds(off * LANES, LANES)] = ((s >> 8) << 7) | (s & 127)
        oidx_v[j, pl.ds(off * LANES, LANES)] = ((o >> 8) << 7) | (o & 127)
        spar_v[pl.ds(g * LANES, LANES)] = ((s >> 7) & 1) * EMBED_DIM
        opar_v[pl.ds(g * LANES, LANES)] = ((o >> 7) & 1) * EMBED_DIM

    for h in range(HALVES):
        copies = []
        for j in range(CH_H):
            copies.append(pltpu.async_copy(
                emb2_hbm.at[sidx_v.at[h * CH_H + j]],
                srows_v.at[pl.ds(j * CHUNK, CHUNK)], sem))
            copies.append(pltpu.async_copy(
                emb2_hbm.at[oidx_v.at[h * CH_H + j]],
                orows_v.at[pl.ds(j * CHUNK, CHUNK)], sem))
        for c in copies:
            c.wait()

        def group_body(g, carry):
            loc = g * LANES + lane
            sp = spar_v[pl.ds(h * ROWS_H + g * LANES, LANES)]
            op = opar_v[pl.ds(h * ROWS_H + g * LANES, LANES)]
            acc = jnp.zeros((LANES,), jnp.float32)
            for d in range(EMBED_DIM):
                sv = plsc.load_gather(srows_v, [loc, sp + d])
                ov = plsc.load_gather(orows_v, [loc, op + d])
                acc = acc + sv * ov
            out_v[pl.ds(h * ROWS_H + g * LANES, LANES)] = acc
            return carry

        lax.fori_loop(0, GROUPS_H, group_body, 0)

    pltpu.sync_copy(out_v, out_hbm.at[pl.ds(base, BPW)])


def kernel(triples, entity_embedding):
    emb2 = _relayout(entity_embedding.T)
    return _score_kernel(triples, emb2)


# speedup vs baseline: 30.7330x; 1.0524x over previous
import functools

import jax
import jax.numpy as jnp
from jax import lax
from jax.experimental import pallas as pl
from jax.experimental.pallas import tpu as pltpu
from jax.experimental.pallas import tpu_sc as plsc

NUM_NODES = 1000000
EMBED_DIM = 64
BATCH = 16384
PAIR = 2 * EMBED_DIM

NC = 2
NS = 16
LANES = 16
NW = NC * NS
BPW = BATCH // NW
CHUNK = 128
NCHUNK = BPW // CHUNK
HALVES = 2
ROWS_H = BPW // HALVES
CH_H = NCHUNK // HALVES
GROUPS_H = ROWS_H // LANES

ABLK = 32768
ASTEPS = -(-NUM_NODES // ABLK)
OUT_ROWS = ASTEPS * (ABLK // 2)


def _relayout_body(x_ref, out_ref):
    x = x_ref[...]
    t = x.T
    packed = [
        jnp.concatenate([t[a * 256:a * 256 + 128],
                         t[a * 256 + 128:a * 256 + 256]], axis=1)
        for a in range(ABLK // 256)
    ]
    out_ref[...] = jnp.concatenate(packed, axis=0)


_relayout = pl.pallas_call(
    _relayout_body,
    grid=(ASTEPS,),
    in_specs=[
        pl.BlockSpec((EMBED_DIM, ABLK), lambda i: (0, i)),
    ],
    out_specs=pl.BlockSpec((ABLK // 2, PAIR), lambda i: (i, 0)),
    out_shape=jax.ShapeDtypeStruct((OUT_ROWS, PAIR), jnp.float32),
)

_mesh = plsc.VectorSubcoreMesh(
    core_axis_name="c", subcore_axis_name="s", num_cores=NC, num_subcores=NS
)


@functools.partial(
    pl.kernel,
    out_type=jax.ShapeDtypeStruct((BATCH,), jnp.float32),
    mesh=_mesh,
    scratch_types=[
        pltpu.VMEM((BPW, 3), jnp.int32),
        pltpu.VMEM((NCHUNK, CHUNK), jnp.int32),
        pltpu.VMEM((NCHUNK, CHUNK), jnp.int32),
        pltpu.VMEM((BPW,), jnp.int32),
        pltpu.VMEM((BPW,), jnp.int32),
        pltpu.VMEM((ROWS_H, PAIR), jnp.float32),
        pltpu.VMEM((ROWS_H, PAIR), jnp.float32),
        pltpu.VMEM((BPW,), jnp.float32),
        pltpu.SemaphoreType.DMA,
    ],
    compiler_params=pltpu.CompilerParams(
        needs_layout_passes=False, use_tc_tiling_on_sc=False),
)
def _score_kernel(tri_hbm, emb2_hbm, out_hbm,
                  tri_v, sidx_v, oidx_v, spar_v, opar_v,
                  srows_v, orows_v, out_v, sem):
    wid = lax.axis_index("s") * NC + lax.axis_index("c")
    base = wid * BPW

    pltpu.sync_copy(tri_hbm.at[pl.ds(base, BPW)], tri_v)

    lane = jnp.arange(LANES, dtype=jnp.int32)
    col0 = jnp.zeros((LANES,), jnp.int32)
    col1 = jnp.ones((LANES,), jnp.int32)

    for g in range(BPW // LANES):
        rows = g * LANES + lane
        s = plsc.load_gather(tri_v, [rows, col0])
        o = plsc.load_gather(tri_v, [rows, col1])
        j, off = divmod(g, CHUNK // LANES)
        sidx_v[j, pl.ds(off * LANES, LANES)] = ((s >> 8) << 7) | (s & 127)
        oidx_v[j, pl.ds(off * LANES, LANES)] = ((o >> 8) << 7) | (o & 127)
        spar_v[pl.ds(g * LANES, LANES)] = ((s >> 7) & 1) * EMBED_DIM
        opar_v[pl.ds(g * LANES, LANES)] = ((o >> 7) & 1) * EMBED_DIM

    for h in range(HALVES):
        copies = []
        for j in range(CH_H):
            copies.append(pltpu.async_copy(
                emb2_hbm.at[sidx_v.at[h * CH_H + j]],
                srows_v.at[pl.ds(j * CHUNK, CHUNK)], sem))
            copies.append(pltpu.async_copy(
                emb2_hbm.at[oidx_v.at[h * CH_H + j]],
                orows_v.at[pl.ds(j * CHUNK, CHUNK)], sem))
        for c in copies:
            c.wait()

        def group_body(g, carry):
            loc = g * LANES + lane
            sp = spar_v[pl.ds(h * ROWS_H + g * LANES, LANES)]
            op = opar_v[pl.ds(h * ROWS_H + g * LANES, LANES)]
            acc = jnp.zeros((LANES,), jnp.float32)
            for d in range(EMBED_DIM):
                sv = plsc.load_gather(srows_v, [loc, sp + d])
                ov = plsc.load_gather(orows_v, [loc, op + d])
                acc = acc + sv * ov
            out_v[pl.ds(h * ROWS_H + g * LANES, LANES)] = acc
            return carry

        lax.fori_loop(0, GROUPS_H, group_body, 0)

    pltpu.sync_copy(out_v, out_hbm.at[pl.ds(base, BPW)])


def kernel(triples, entity_embedding):
    emb2 = _relayout(entity_embedding.T)
    return _score_kernel(triples, emb2)
